# baseline (speedup 1.0000x reference)
import functools

import jax
import jax.numpy as jnp
from jax import lax
from jax.experimental import pallas as pl
from jax.experimental.pallas import tpu as pltpu
from jax.experimental.pallas import tpu_sc as plsc

B, S = 2, 2048
H = 2048
NUM_CB = 32
AV3 = 2054
P = B * S
NW = 32
PPW = P // NW
BLK = 8
NBLK = PPW // BLK
IDS_PAD = 64
LANES = 16

_mesh = plsc.VectorSubcoreMesh(core_axis_name="c", subcore_axis_name="s")


@functools.partial(
    pl.kernel,
    out_type=jax.ShapeDtypeStruct((P, H), jnp.float32),
    mesh=_mesh,
    compiler_params=pltpu.CompilerParams(needs_layout_passes=False),
    scratch_types=[
        pltpu.VMEM((BLK * IDS_PAD,), jnp.int32),
        pltpu.VMEM((NUM_CB,), jnp.int32),
        pltpu.VMEM((NUM_CB,), jnp.float32),
        pltpu.VMEM((LANES,), jnp.int32),
        pltpu.VMEM((LANES,), jnp.float32),
        pltpu.VMEM((NUM_CB, H), jnp.float32),
        pltpu.VMEM((LANES, H), jnp.float32),
        pltpu.VMEM((BLK, H), jnp.float32),
        pltpu.SemaphoreType.DMA,
        pltpu.SemaphoreType.DMA,
    ],
)
def _embed_kernel(ids_hbm, text_hbm, audio_hbm, out_hbm,
                  tok_v, aidx_v, amask_v, tidx_v, tmask_v,
                  arows_v, trows_v, acc_v, sem_a, sem_t):
    wid = lax.axis_index("s") * 2 + lax.axis_index("c")
    base = wid * PPW
    lanes = lax.broadcasted_iota(jnp.int32, (LANES,), 0)

    def block_body(blk, _):
        row0 = base + blk * BLK
        pltpu.sync_copy(
            ids_hbm.at[pl.ds(row0 * IDS_PAD, BLK * IDS_PAD)], tok_v)

        rowsel = jnp.minimum(lanes, BLK - 1)
        ttok = plsc.load_gather(tok_v, [rowsel * IDS_PAD + NUM_CB])
        tidx_v[...] = ttok
        tmask_v[...] = jnp.where(ttok != 0, 1.0, 0.0)
        tcopy = pltpu.async_copy(text_hbm.at[tidx_v], trows_v, sem_t)
        tcopy.wait()

        def text_init(p, _):
            scale = plsc.load_gather(tmask_v, [jnp.full((LANES,), p, jnp.int32)])

            def jbody(j, _):
                acc_v[p, pl.ds(j * LANES, LANES)] = (
                    trows_v[p, pl.ds(j * LANES, LANES)] * scale)
                return 0

            lax.fori_loop(0, H // LANES, jbody, 0)
            return 0

        lax.fori_loop(0, BLK, text_init, 0)

        def pos_body(p, _):
            atok_lo = tok_v[pl.ds(p * IDS_PAD, LANES)]
            atok_hi = tok_v[pl.ds(p * IDS_PAD + LANES, LANES)]
            aidx_v[pl.ds(0, LANES)] = atok_lo + lanes * AV3
            aidx_v[pl.ds(LANES, LANES)] = atok_hi + (lanes + LANES) * AV3
            amask_v[pl.ds(0, LANES)] = jnp.where(atok_lo != 0, 1.0, 0.0)
            amask_v[pl.ds(LANES, LANES)] = jnp.where(atok_hi != 0, 1.0, 0.0)
            pltpu.async_copy(audio_hbm.at[aidx_v], arows_v, sem_a).wait()

            def row_body(r, _):
                scale = plsc.load_gather(
                    amask_v, [jnp.full((LANES,), r, jnp.int32)])

                def jbody(j, _):
                    plsc.addupdate(
                        acc_v.at[p, pl.ds(j * LANES, LANES)],
                        arows_v[r, pl.ds(j * LANES, LANES)] * scale)
                    return 0

                lax.fori_loop(0, H // LANES, jbody, 0)
                return 0

            lax.fori_loop(0, NUM_CB, row_body, 0)
            return 0

        lax.fori_loop(0, BLK, pos_body, 0)
        pltpu.sync_copy(acc_v, out_hbm.at[pl.ds(row0, BLK), :])
        return 0

    lax.fori_loop(0, NBLK, block_body, 0)


def kernel(input_ids, text_table, audio_table):
    ids = input_ids.reshape(P, NUM_CB + 1).astype(jnp.int32)
    ids_pad = jnp.pad(ids, ((0, 0), (0, IDS_PAD - (NUM_CB + 1)))).reshape(-1)
    out = _embed_kernel(ids_pad, text_table, audio_table)
    return out.reshape(B, S, H)


# speedup vs baseline: 5.2085x; 5.2085x over previous
import functools

import jax
import jax.numpy as jnp
from jax import lax
from jax.experimental import pallas as pl
from jax.experimental.pallas import tpu as pltpu
from jax.experimental.pallas import tpu_sc as plsc

B, S = 2, 2048
H = 2048
NUM_CB = 32
AV3 = 2054
P = B * S
NW = 32
PPW = P // NW
BLK = 8
NBLK = PPW // BLK
IDS_PAD = 64
LANES = 16
HBLK = 256
NHB = H // HBLK
NACC = HBLK // LANES


_mesh = plsc.VectorSubcoreMesh(core_axis_name="c", subcore_axis_name="s")


@functools.partial(
    pl.kernel,
    out_type=jax.ShapeDtypeStruct((P, H), jnp.float32),
    mesh=_mesh,
    compiler_params=pltpu.CompilerParams(needs_layout_passes=False),
    scratch_types=[
        pltpu.VMEM((PPW * IDS_PAD,), jnp.int32),
        pltpu.VMEM((2, LANES), jnp.int32),
        pltpu.VMEM((NUM_CB,), jnp.float32),
        pltpu.VMEM((LANES,), jnp.int32),
        pltpu.VMEM((LANES,), jnp.float32),
        pltpu.VMEM((2, LANES, H), jnp.float32),
        pltpu.VMEM((LANES, H), jnp.float32),
        pltpu.VMEM((BLK, H), jnp.float32),
        pltpu.SemaphoreType.DMA,
        pltpu.SemaphoreType.DMA,
        pltpu.SemaphoreType.DMA,
    ],
)
def _embed_kernel(ids_hbm, text_hbm, audio_hbm, out_hbm,
                  tok_v, aidx_v, amask_v, tidx_v, tmask_v,
                  arows_v, trows_v, acc_v, sem0, sem1, sem_t):
    wid = lax.axis_index("s") * 2 + lax.axis_index("c")
    base = wid * PPW
    lanes = lax.broadcasted_iota(jnp.int32, (LANES,), 0)
    sems = (sem0, sem1)

    pltpu.sync_copy(ids_hbm.at[pl.ds(base * IDS_PAD, PPW * IDS_PAD)], tok_v)

    def splat(ref, i):
        return plsc.load_gather(ref, [jnp.full((LANES,), i, jnp.int32)])

    def start_half(gp, half, slot):
        atok = tok_v[pl.ds(gp * IDS_PAD + half * LANES, LANES)]
        aidx_v[slot, :] = atok + (lanes + half * LANES) * AV3
        pltpu.async_copy(
            audio_hbm.at[aidx_v.at[slot]], arows_v.at[slot], sems[slot])

    def wait_slot(slot):
        pltpu.make_async_copy(
            audio_hbm.at[aidx_v.at[slot]], arows_v.at[slot],
            sems[slot]).wait()

    def block_body(blk, _):
        row0 = base + blk * BLK

        rowsel = blk * BLK + jnp.minimum(lanes, BLK - 1)
        ttok = plsc.load_gather(tok_v, [rowsel * IDS_PAD + NUM_CB])
        tidx_v[...] = ttok
        tmask_v[...] = jnp.where(ttok != 0, 1.0, 0.0)
        tcopy = pltpu.async_copy(text_hbm.at[tidx_v], trows_v, sem_t)

        start_half(blk * BLK, 0, 0)
        tcopy.wait()

        def pos_body(p, _):
            pbase = (blk * BLK + p) * IDS_PAD
            atok_lo = tok_v[pl.ds(pbase, LANES)]
            atok_hi = tok_v[pl.ds(pbase + LANES, LANES)]
            amask_v[pl.ds(0, LANES)] = jnp.where(atok_lo != 0, 1.0, 0.0)
            amask_v[pl.ds(LANES, LANES)] = jnp.where(atok_hi != 0, 1.0, 0.0)
            tscale = splat(tmask_v, p)

            for half in (0, 1):
                slot = half
                wait_slot(slot)
                if half == 0:
                    start_half(blk * BLK + p, 1, 1)
                else:
                    @pl.when(p + 1 < BLK)
                    def _():
                        start_half(blk * BLK + p + 1, 0, 0)

                rows = arows_v.at[slot]

                def hb_body(hb, _):
                    hoff = hb * HBLK
                    if half == 0:
                        accs = [
                            trows_v[p, pl.ds(hoff + k * LANES, LANES)] * tscale
                            for k in range(NACC)
                        ]
                    else:
                        accs = [
                            acc_v[p, pl.ds(hoff + k * LANES, LANES)]
                            for k in range(NACC)
                        ]

                    def r_body(r, accs):
                        scale = splat(amask_v, half * LANES + r)
                        return [
                            a + rows[r, pl.ds(hoff + k * LANES, LANES)] * scale
                            for k, a in enumerate(accs)
                        ]

                    accs = lax.fori_loop(0, LANES, r_body, accs)
                    for k in range(NACC):
                        acc_v[p, pl.ds(hoff + k * LANES, LANES)] = accs[k]
                    return 0

                lax.fori_loop(0, NHB, hb_body, 0)
            return 0

        lax.fori_loop(0, BLK, pos_body, 0)
        pltpu.sync_copy(acc_v, out_hbm.at[pl.ds(row0, BLK), :])
        return 0

    lax.fori_loop(0, NBLK, block_body, 0)


def kernel(input_ids, text_table, audio_table):
    ids = input_ids.reshape(P, NUM_CB + 1).astype(jnp.int32)
    ids_pad = jnp.pad(ids, ((0, 0), (0, IDS_PAD - (NUM_CB + 1)))).reshape(-1)
    out = _embed_kernel(ids_pad, text_table, audio_table)
    return out.reshape(B, S, H)


# speedup vs baseline: 5.5905x; 1.0733x over previous
import functools

import jax
import jax.numpy as jnp
from jax import lax
from jax.experimental import pallas as pl
from jax.experimental.pallas import tpu as pltpu
from jax.experimental.pallas import tpu_sc as plsc

B, S = 2, 2048
H = 2048
NUM_CB = 32
AV3 = 2054
P = B * S
NW = 32
PPW = P // NW
BLK = 8
NBLK = PPW // BLK
IDS_PAD = 64
LANES = 16
HBLK = 256
NHB = H // HBLK
NACC = HBLK // LANES


_mesh = plsc.VectorSubcoreMesh(core_axis_name="c", subcore_axis_name="s")


@functools.partial(
    pl.kernel,
    out_type=jax.ShapeDtypeStruct((P, H), jnp.float32),
    mesh=_mesh,
    compiler_params=pltpu.CompilerParams(needs_layout_passes=False),
    scratch_types=[
        pltpu.VMEM((PPW * IDS_PAD,), jnp.int32),
        pltpu.VMEM((2, LANES), jnp.int32),
        pltpu.VMEM((NUM_CB,), jnp.float32),
        pltpu.VMEM((LANES,), jnp.int32),
        pltpu.VMEM((LANES,), jnp.float32),
        pltpu.VMEM((2, LANES, H), jnp.float32),
        pltpu.VMEM((LANES, H), jnp.float32),
        pltpu.VMEM((BLK, H), jnp.float32),
        pltpu.SemaphoreType.DMA,
        pltpu.SemaphoreType.DMA,
        pltpu.SemaphoreType.DMA,
    ],
)
def _embed_kernel(ids_hbm, text_hbm, audio_hbm, out_hbm,
                  tok_v, aidx_v, amask_v, tidx_v, tmask_v,
                  arows_v, trows_v, acc_v, sem0, sem1, sem_t):
    wid = lax.axis_index("s") * 2 + lax.axis_index("c")
    base = wid * PPW
    lanes = lax.broadcasted_iota(jnp.int32, (LANES,), 0)
    sems = (sem0, sem1)

    pltpu.sync_copy(ids_hbm.at[pl.ds(base * IDS_PAD, PPW * IDS_PAD)], tok_v)

    def splat(ref, i):
        return plsc.load_gather(ref, [jnp.full((LANES,), i, jnp.int32)])

    def start_half(gp, half, slot):
        atok = tok_v[pl.ds(gp * IDS_PAD + half * LANES, LANES)]
        aidx_v[slot, :] = atok + (lanes + half * LANES) * AV3
        pltpu.async_copy(
            audio_hbm.at[aidx_v.at[slot]], arows_v.at[slot], sems[slot])

    def wait_slot(slot):
        pltpu.make_async_copy(
            audio_hbm.at[aidx_v.at[slot]], arows_v.at[slot],
            sems[slot]).wait()

    def pair_body(bp, _):
        pair0 = bp * (2 * BLK)

        rowsel = pair0 + lanes
        ttok = plsc.load_gather(tok_v, [rowsel * IDS_PAD + NUM_CB])
        tidx_v[...] = ttok
        tmask_v[...] = jnp.where(ttok != 0, 1.0, 0.0)
        tcopy = pltpu.async_copy(text_hbm.at[tidx_v], trows_v, sem_t)

        start_half(pair0, 0, 0)
        tcopy.wait()

        for sub in (0, 1):
            sub0 = pair0 + sub * BLK

            def pos_body(p, _):
                pbase = (sub0 + p) * IDS_PAD
                atok_lo = tok_v[pl.ds(pbase, LANES)]
                atok_hi = tok_v[pl.ds(pbase + LANES, LANES)]
                amask_v[pl.ds(0, LANES)] = jnp.where(atok_lo != 0, 1.0, 0.0)
                amask_v[pl.ds(LANES, LANES)] = jnp.where(atok_hi != 0, 1.0, 0.0)
                tscale = splat(tmask_v, sub * BLK + p)

                for half in (0, 1):
                    slot = half
                    wait_slot(slot)
                    if half == 0:
                        start_half(sub0 + p, 1, 1)
                    elif sub == 0:
                        start_half(sub0 + p + 1, 0, 0)
                    else:
                        @pl.when(p + 1 < BLK)
                        def _():
                            start_half(sub0 + p + 1, 0, 0)

                    rows = arows_v.at[slot]

                    def hb_body(hb, _):
                        hoff = hb * HBLK
                        if half == 0:
                            accs = [
                                trows_v[sub * BLK + p,
                                        pl.ds(hoff + k * LANES, LANES)] * tscale
                                for k in range(NACC)
                            ]
                        else:
                            accs = [
                                acc_v[p, pl.ds(hoff + k * LANES, LANES)]
                                for k in range(NACC)
                            ]

                        def r_body(r, accs):
                            scale = splat(amask_v, half * LANES + r)
                            return [
                                a + rows[r, pl.ds(hoff + k * LANES, LANES)]
                                * scale
                                for k, a in enumerate(accs)
                            ]

                        accs = lax.fori_loop(0, LANES, r_body, accs)
                        for k in range(NACC):
                            acc_v[p, pl.ds(hoff + k * LANES, LANES)] = accs[k]
                        return 0

                    lax.fori_loop(0, NHB, hb_body, 0)
                return 0

            lax.fori_loop(0, BLK, pos_body, 0)
            pltpu.sync_copy(acc_v, out_hbm.at[pl.ds(base + sub0, BLK), :])
        return 0

    lax.fori_loop(0, NBLK // 2, pair_body, 0)


def kernel(input_ids, text_table, audio_table):
    ids = input_ids.reshape(P, NUM_CB + 1).astype(jnp.int32)
    ids_pad = jnp.pad(ids, ((0, 0), (0, IDS_PAD - (NUM_CB + 1)))).reshape(-1)
    out = _embed_kernel(ids_pad, text_table, audio_table)
    return out.reshape(B, S, H)
